# baseline (speedup 1.0000x reference)
import functools

import jax
import jax.numpy as jnp
from jax.experimental import pallas as pl

_F8 = jnp.float8_e4m3fn


def _xw_body(x_ref, w_ref, o_ref):
    o_ref[...] = jnp.dot(x_ref[...], w_ref[...],
                         preferred_element_type=jnp.float32
                         ).astype(jnp.bfloat16)


def _pass1_body(adj_ref, y_ref, b_ref, w_ref, adj8_ref, ynext_ref):
    a = adj_ref[...]
    h = jnp.maximum(
        jnp.dot(a.astype(jnp.bfloat16), y_ref[...],
                preferred_element_type=jnp.float32)
        + b_ref[...], 0.0)
    adj8_ref[...] = a.astype(_F8)
    ynext_ref[...] = jnp.dot(h, w_ref[...],
                             preferred_element_type=jnp.float32
                             ).astype(jnp.bfloat16)


def _mid_body(adj_ref, y_ref, b_ref, w_ref, ynext_ref):
    a16 = adj_ref[...].astype(jnp.bfloat16)
    h = jnp.maximum(
        jnp.dot(a16, y_ref[...], preferred_element_type=jnp.float32)
        + b_ref[...], 0.0)
    ynext_ref[...] = jnp.dot(h, w_ref[...],
                             preferred_element_type=jnp.float32
                             ).astype(jnp.bfloat16)


def _last_body(adj_ref, y_ref, b_ref, out_ref):
    a16 = adj_ref[...].astype(jnp.bfloat16)
    out_ref[...] = (
        jnp.dot(a16, y_ref[...], preferred_element_type=jnp.float32)
        + b_ref[...])


def kernel(x, adj, W1, b1, W2, b2, W3, b3, W4, b4):
    n, nfeat = x.shape
    h1 = W1.shape[1]
    h2 = W2.shape[1]
    h3 = W3.shape[1]
    ncls = W4.shape[1]
    bm1 = 256
    bm2 = 512

    strip = lambda bm, width: pl.BlockSpec((bm, width), lambda i: (i, 0))
    whole = lambda shp: pl.BlockSpec(shp, lambda i: (0, 0))

    y1 = pl.pallas_call(
        _xw_body,
        out_shape=jax.ShapeDtypeStruct((n, h1), jnp.bfloat16),
    )(x, W1)

    adj8, y2 = pl.pallas_call(
        _pass1_body,
        grid=(pl.cdiv(n, bm1),),
        in_specs=[strip(bm1, n), whole((n, h1)), whole((1, h1)),
                  whole((h1, h2))],
        out_specs=[strip(bm1, n), strip(bm1, h2)],
        out_shape=[jax.ShapeDtypeStruct((n, n), _F8),
                   jax.ShapeDtypeStruct((n, h2), jnp.bfloat16)],
    )(adj, y1, b1.reshape(1, h1), W2)

    y3 = pl.pallas_call(
        _mid_body,
        grid=(pl.cdiv(n, bm2),),
        in_specs=[strip(bm2, n), whole((n, h2)), whole((1, h2)),
                  whole((h2, h3))],
        out_specs=strip(bm2, h3),
        out_shape=jax.ShapeDtypeStruct((n, h3), jnp.bfloat16),
    )(adj8, y2, b2.reshape(1, h2), W3)

    y4 = pl.pallas_call(
        _mid_body,
        grid=(pl.cdiv(n, bm2),),
        in_specs=[strip(bm2, n), whole((n, h3)), whole((1, h3)),
                  whole((h3, ncls))],
        out_specs=strip(bm2, ncls),
        out_shape=jax.ShapeDtypeStruct((n, ncls), jnp.bfloat16),
    )(adj8, y3, b3.reshape(1, h3), W4)

    out = pl.pallas_call(
        _last_body,
        grid=(pl.cdiv(n, bm2),),
        in_specs=[strip(bm2, n), whole((n, ncls)), whole((1, ncls))],
        out_specs=strip(bm2, ncls),
        out_shape=jax.ShapeDtypeStruct((n, ncls), jnp.float32),
    )(adj8, y4, b4.reshape(1, ncls))

    return out
